# Initial kernel scaffold; baseline (speedup 1.0000x reference)
import jax
import jax.numpy as jnp
from jax.experimental import pallas as pl


def kernel(text_1, text_2, numeric_features, table, W1, b1, W2, b2):
    raise NotImplementedError("write your pallas kernel here")


# speedup vs baseline: 2.1578x; 2.1578x over previous
import jax
import jax.numpy as jnp
from jax import lax
from jax.experimental import pallas as pl
from jax.experimental.pallas import tpu as pltpu
from jax.experimental.pallas import tpu_sc as plsc

_B = 16384
_D = 10
_DP = 16
_C1 = 1
_C2 = 3

_NC = 2
_NS = 16
_NW = _NC * _NS
_BPW = _B // _NW
_CHUNK = 128
_NCHUNK = _BPW // _CHUNK


def _sc_gather_body(t1_hbm, t2_hbm, table_hbm, o1_hbm, o2_hbm,
                    idx1_v, idx2_v, rows1_v, rows2_v, sem):
  wid = lax.axis_index("s") * _NC + lax.axis_index("c")
  base = wid * _BPW
  pltpu.sync_copy(t1_hbm.at[pl.ds(base, _BPW)], idx1_v)
  pltpu.sync_copy(t2_hbm.at[pl.ds(base, _BPW)], idx2_v)
  copies = []
  for j in range(_NCHUNK):
    sl = pl.ds(j * _CHUNK, _CHUNK)
    copies.append(pltpu.async_copy(table_hbm.at[idx1_v.at[sl]], rows1_v.at[sl], sem))
    copies.append(pltpu.async_copy(table_hbm.at[idx2_v.at[sl]], rows2_v.at[sl], sem))
  for cp in copies:
    cp.wait()
  pltpu.sync_copy(rows1_v, o1_hbm.at[pl.ds(base, _BPW)])
  pltpu.sync_copy(rows2_v, o2_hbm.at[pl.ds(base, _BPW)])


def _sc_gather(t1, t2, table_padded):
  mesh = plsc.VectorSubcoreMesh(core_axis_name="c", subcore_axis_name="s")
  fn = pl.kernel(
      _sc_gather_body,
      out_type=[
          jax.ShapeDtypeStruct((_B, _DP), jnp.float32),
          jax.ShapeDtypeStruct((_B, _DP), jnp.float32),
      ],
      mesh=mesh,
      scratch_types=[
          pltpu.VMEM((_BPW,), jnp.int32),
          pltpu.VMEM((_BPW,), jnp.int32),
          pltpu.VMEM((_BPW, _DP), jnp.float32),
          pltpu.VMEM((_BPW, _DP), jnp.float32),
          pltpu.SemaphoreType.DMA,
      ],
      compiler_params=pltpu.CompilerParams(use_tc_tiling_on_sc=False),
  )
  return fn(t1, t2, table_padded)


def _dense_body(e1p_ref, e2p_ref, num_ref, w1a_ref, w1b_ref, w1c_ref,
                b1_ref, w2_ref, b2_ref, out_ref):
  e1p = e1p_ref[:]
  e2p = e2p_ref[:]
  ws1 = e1p[0:_B] + e1p[1:_B + 1] + e1p[2:_B + 2]
  ws2 = e2p[0:_B]
  for o in range(1, 2 * _C2 + 1):
    ws2 = ws2 + e2p[o:o + _B]
  i = lax.broadcasted_iota(jnp.int32, (_B, 1), 0)
  d1 = (1 + jnp.minimum(i, _C1) + jnp.minimum(_B - 1 - i, _C1)).astype(jnp.float32)
  d2 = (1 + jnp.minimum(i, _C2) + jnp.minimum(_B - 1 - i, _C2)).astype(jnp.float32)
  e1m = ws1 / d1
  e2m = ws2 / d2
  h = (jnp.dot(e1m, w1a_ref[:], preferred_element_type=jnp.float32)
       + jnp.dot(e2m, w1b_ref[:], preferred_element_type=jnp.float32)
       + jnp.dot(num_ref[:], w1c_ref[:], preferred_element_type=jnp.float32)
       + b1_ref[:])
  h = jnp.maximum(h, 0.0)
  logits = jnp.dot(h, w2_ref[:], preferred_element_type=jnp.float32) + b2_ref[:]
  out_ref[:] = jax.nn.sigmoid(logits)


def _dense(e1p, e2p, numeric, w1a, w1b, w1c, b1, W2, b2, interpret=False):
  return pl.pallas_call(
      _dense_body,
      out_shape=jax.ShapeDtypeStruct((_B, 1), jnp.float32),
      interpret=interpret,
  )(e1p, e2p, numeric, w1a, w1b, w1c, b1.reshape(1, -1), W2, b2.reshape(1, 1))


def kernel(text_1, text_2, numeric_features, table, W1, b1, W2, b2):
  t1 = text_1.astype(jnp.int32)
  t2 = text_2.astype(jnp.int32)
  table_padded = jnp.pad(table, ((0, 0), (0, _DP - _D)))
  g1, g2 = _sc_gather(t1, t2, table_padded)
  e1p = jnp.pad(g1, ((_C1, _C1), (0, 0)))
  e2p = jnp.pad(g2, ((_C2, _C2), (0, 0)))
  w1a = jnp.pad(W1[0:_D], ((0, _DP - _D), (0, 0)))
  w1b = jnp.pad(W1[_D:2 * _D], ((0, _DP - _D), (0, 0)))
  w1c = W1[2 * _D:]
  return _dense(e1p, e2p, numeric_features, w1a, w1b, w1c, b1, W2, b2)
